# baseline (speedup 1.0000x reference)
import jax
import jax.numpy as jnp
from jax.experimental import pallas as pl
from jax.experimental.pallas import tpu as pltpu

_B, _D, _T = 16, 256, 2048
_K, _Q = 1024, 8
_N = _B * _T
_BLK = 512
_NCH = 2
_H = _BLK // _NCH


def _vq_body(x_ref, cbt_ref, hi_ref, mid_ref, cbsq_ref,
             qt_ref, codes_ref):
    iota_f = jax.lax.broadcasted_iota(jnp.int32, (_H, _K), 1).astype(jnp.float32)

    rs = [x_ref[h * _H:(h + 1) * _H, :] for h in range(_NCH)]
    qts = [jnp.zeros_like(rs[h]) for h in range(_NCH)]
    for q in range(_Q):
        for h in range(_NCH):
            r = rs[h]
            s = jnp.sum(r * r, axis=1, keepdims=True)
            m = jnp.dot(r, cbt_ref[q], preferred_element_type=jnp.float32)
            dist = s - 2.0 * m + cbsq_ref[q][None, :]
            mn = jnp.min(dist, axis=1, keepdims=True)
            idxf = jnp.min(jnp.where(dist == mn, iota_f, float(_K)), axis=1)
            sel = (iota_f == idxf[:, None]).astype(jnp.bfloat16)
            quant = (jnp.dot(sel, hi_ref[q], preferred_element_type=jnp.float32)
                     + jnp.dot(sel, mid_ref[q], preferred_element_type=jnp.float32))
            rs[h] = r - quant
            qts[h] = qts[h] + quant
            codes_ref[h * _H:(h + 1) * _H, q] = idxf.astype(jnp.int32)
    for h in range(_NCH):
        qt_ref[h * _H:(h + 1) * _H, :] = qts[h]


def kernel(embeddings, embed):
    x = jnp.transpose(embeddings, (0, 2, 1)).reshape(-1, _D)
    cbt = jnp.transpose(embed, (0, 2, 1))
    hi = jax.lax.optimization_barrier(embed.astype(jnp.bfloat16))
    r1 = embed - hi.astype(jnp.float32)
    mid = jax.lax.optimization_barrier(r1.astype(jnp.bfloat16))
    cbsq = jnp.sum(embed * embed, axis=2)
    grid = (_N // _BLK,)
    qt, codes = pl.pallas_call(
        _vq_body,
        grid=grid,
        in_specs=[
            pl.BlockSpec((_BLK, _D), lambda i: (i, 0)),
            pl.BlockSpec((_Q, _D, _K), lambda i: (0, 0, 0)),
            pl.BlockSpec((_Q, _K, _D), lambda i: (0, 0, 0)),
            pl.BlockSpec((_Q, _K, _D), lambda i: (0, 0, 0)),
            pl.BlockSpec((_Q, _K), lambda i: (0, 0)),
        ],
        out_specs=[
            pl.BlockSpec((_BLK, _D), lambda i: (i, 0)),
            pl.BlockSpec((_BLK, _Q), lambda i: (i, 0)),
        ],
        out_shape=[
            jax.ShapeDtypeStruct((_N, _D), jnp.float32),
            jax.ShapeDtypeStruct((_N, _Q), jnp.int32),
        ],
    )(x, cbt, hi, mid, cbsq)

    quantized_out = jnp.transpose(qt.reshape(_B, _T, _D), (0, 2, 1))
    return (quantized_out, jnp.transpose(codes).reshape(_Q, _B, _T))


# speedup vs baseline: 1.1289x; 1.1289x over previous
import jax
import jax.numpy as jnp
from jax.experimental import pallas as pl
from jax.experimental.pallas import tpu as pltpu

_B, _D, _T = 16, 256, 2048
_K, _Q = 1024, 8
_N = _B * _T
_BLK = 1024
_NCH = 2
_H = _BLK // _NCH


def _vq_body(x_ref, cbt_ref, hi_ref, mid_ref, cbsq_ref,
             qt_ref, codes_ref):
    iota_f = jax.lax.broadcasted_iota(jnp.int32, (_H, _K), 1).astype(jnp.float32)

    rs = [x_ref[h * _H:(h + 1) * _H, :] for h in range(_NCH)]
    qts = [jnp.zeros_like(rs[h]) for h in range(_NCH)]
    for q in range(_Q):
        for h in range(_NCH):
            r = rs[h]
            s = jnp.sum(r * r, axis=1, keepdims=True)
            m = jnp.dot(r, cbt_ref[q], preferred_element_type=jnp.float32)
            dist = s - 2.0 * m + cbsq_ref[q][None, :]
            mn = jnp.min(dist, axis=1, keepdims=True)
            idxf = jnp.min(jnp.where(dist == mn, iota_f, float(_K)), axis=1)
            sel = (iota_f == idxf[:, None]).astype(jnp.bfloat16)
            quant = (jnp.dot(sel, hi_ref[q], preferred_element_type=jnp.float32)
                     + jnp.dot(sel, mid_ref[q], preferred_element_type=jnp.float32))
            rs[h] = r - quant
            qts[h] = qts[h] + quant
            codes_ref[h * _H:(h + 1) * _H, q] = idxf.astype(jnp.int32)
    for h in range(_NCH):
        qt_ref[h * _H:(h + 1) * _H, :] = qts[h]


def kernel(embeddings, embed):
    x = jnp.transpose(embeddings, (0, 2, 1)).reshape(-1, _D)
    cbt = jnp.transpose(embed, (0, 2, 1)).astype(jnp.bfloat16)
    hi = jax.lax.optimization_barrier(embed.astype(jnp.bfloat16))
    r1 = embed - hi.astype(jnp.float32)
    mid = jax.lax.optimization_barrier(r1.astype(jnp.bfloat16))
    cbsq = jnp.sum(embed * embed, axis=2)
    grid = (_N // _BLK,)
    qt, codes = pl.pallas_call(
        _vq_body,
        grid=grid,
        in_specs=[
            pl.BlockSpec((_BLK, _D), lambda i: (i, 0)),
            pl.BlockSpec((_Q, _D, _K), lambda i: (0, 0, 0)),
            pl.BlockSpec((_Q, _K, _D), lambda i: (0, 0, 0)),
            pl.BlockSpec((_Q, _K, _D), lambda i: (0, 0, 0)),
            pl.BlockSpec((_Q, _K), lambda i: (0, 0)),
        ],
        out_specs=[
            pl.BlockSpec((_BLK, _D), lambda i: (i, 0)),
            pl.BlockSpec((_BLK, _Q), lambda i: (i, 0)),
        ],
        out_shape=[
            jax.ShapeDtypeStruct((_N, _D), jnp.float32),
            jax.ShapeDtypeStruct((_N, _Q), jnp.int32),
        ],
    )(x, cbt, hi, mid, cbsq)

    quantized_out = jnp.transpose(qt.reshape(_B, _T, _D), (0, 2, 1))
    return (quantized_out, jnp.transpose(codes).reshape(_Q, _B, _T))
